# baseline (speedup 1.0000x reference)
import functools

import jax
import jax.numpy as jnp
from jax import lax
from jax.experimental import pallas as pl
from jax.experimental.pallas import tpu as pltpu
from jax.experimental.pallas import tpu_sc as plsc

B, SL, DL, D = 1024, 200, 200, 64
L2 = SL + DL
NP = L2 // 2
LP = 512
LD = 1024
NC, NS = 2, 16
NW = NC * NS
ROWS_PER_W = B // NW
HSIZE = 100000
RB = 8

_mesh = plsc.VectorSubcoreMesh(core_axis_name="c", subcore_axis_name="s",
                               num_cores=NC, num_subcores=NS)


def _sc_count_body(ids_hbm, idsd_hbm, wvec_hbm, zvec_hbm, a_hbm,
                   hist, ids_v, idsd_v, wvec_v, zvec_v, idx_v, idxd_v,
                   cnt_v, a_v):
    c = lax.axis_index("c")
    s = lax.axis_index("s")
    wid = c * NS + s
    pltpu.sync_copy(wvec_hbm, wvec_v)
    pltpu.sync_copy(zvec_hbm, zvec_v)
    m16 = (lax.iota(jnp.int32, 16) & 3) < 2

    def row_step(r, carry):
        row = wid * ROWS_PER_W + r
        pltpu.sync_copy(ids_hbm.at[row], ids_v)
        pltpu.sync_copy(idsd_hbm.at[row], idsd_v)
        for j in range(4):
            for k in range(8):
                sl = pl.ds(k * 16, 16)
                idx_v[j, sl] = ids_v[j, sl] + s * HSIZE
        for j in range(8):
            for k in range(8):
                sl = pl.ds(k * 16, 16)
                idxd_v[j, sl] = idsd_v[j, sl] + s * HSIZE
        for j in range(4):
            pltpu.sync_copy(zvec_v.at[j], hist.at[idx_v.at[j]])
        for j in range(4):
            pltpu.sync_copy(wvec_v.at[j], hist.at[idx_v.at[j]], add=True)
        for j in range(8):
            pltpu.sync_copy(hist.at[idxd_v.at[j]], cnt_v.at[j])
        for j in range(8):
            for k in range(8):
                sl = pl.ds(k * 16, 16)
                cnt16 = cnt_v[j, sl]
                a16 = jnp.where(m16, cnt16 & 0xFFFF, cnt16 >> 16)
                a16 = jnp.where(idsd_v[j, sl] != 0, a16, 0)
                a_v[j, sl] = a16.astype(jnp.float32)
        pltpu.sync_copy(a_v, a_hbm.at[row])
        return carry

    lax.fori_loop(0, ROWS_PER_W, row_step, 0)


@functools.partial(
    pl.kernel,
    out_type=jax.ShapeDtypeStruct((B, 8, 128), jnp.float32),
    mesh=_mesh,
    scratch_types=[
        pltpu.VMEM_SHARED((NS * HSIZE,), jnp.int32),
        pltpu.VMEM((4, 128), jnp.int32),
        pltpu.VMEM((8, 128), jnp.int32),
        pltpu.VMEM((4, 128), jnp.int32),
        pltpu.VMEM((4, 128), jnp.int32),
        pltpu.VMEM((4, 128), jnp.int32),
        pltpu.VMEM((8, 128), jnp.int32),
        pltpu.VMEM((8, 128), jnp.int32),
        pltpu.VMEM((8, 128), jnp.float32),
    ],
)
def _sc_count(*args):
    _sc_count_body(*args)


def _tc_encode_body(a_ref, m1a_ref, m1b_ref, b1p_ref, w2dd_ref, bb2_ref,
                    src_out, dst_out):
    ra = a_ref[:, :NP, :].reshape(RB * NP, 4)
    b1p = b1p_ref[...]
    p0 = jnp.dot(ra, m1a_ref[...], preferred_element_type=jnp.float32) + b1p
    p1 = jnp.dot(ra, m1b_ref[...], preferred_element_type=jnp.float32) + b1p
    h = jnp.maximum(p0, 0.0) + jnp.maximum(p1, 0.0)
    feat = jnp.dot(h, w2dd_ref[...], preferred_element_type=jnp.float32)
    feat = feat + bb2_ref[...]
    feat = feat.reshape(RB, NP, 128)
    src_out[...] = feat[:, :NP // 2, :]
    dst_out[...] = feat[:, NP // 2:, :]


def _tc_encode(a, m1a, m1b, b1p, w2dd, bb2):
    return pl.pallas_call(
        _tc_encode_body,
        grid=(B // RB,),
        in_specs=[
            pl.BlockSpec((RB, 256, 4), lambda i: (i, 0, 0)),
            pl.BlockSpec((4, 128), lambda i: (0, 0)),
            pl.BlockSpec((4, 128), lambda i: (0, 0)),
            pl.BlockSpec((1, 128), lambda i: (0, 0)),
            pl.BlockSpec((128, 128), lambda i: (0, 0)),
            pl.BlockSpec((1, 128), lambda i: (0, 0)),
        ],
        out_specs=[
            pl.BlockSpec((RB, NP // 2, 128), lambda i: (i, 0, 0)),
            pl.BlockSpec((RB, NP // 2, 128), lambda i: (i, 0, 0)),
        ],
        out_shape=[
            jax.ShapeDtypeStruct((B, NP // 2, 128), jnp.float32),
            jax.ShapeDtypeStruct((B, NP // 2, 128), jnp.float32),
        ],
    )(a, m1a, m1b, b1p, w2dd, bb2)


@jax.jit
def kernel(src_ids, dst_ids, W1, b1, W2, b2):
    ids = jnp.concatenate([src_ids.astype(jnp.int32),
                           dst_ids.astype(jnp.int32)], axis=1)
    ids_pad = jnp.pad(ids, ((0, 0), (0, LP - L2))).reshape(B, 4, 128)
    pairs = ids.reshape(B, NP, 2)
    ids_dup = jnp.concatenate([pairs, pairs], axis=2).reshape(B, 2 * L2)
    ids_dup = jnp.pad(ids_dup, ((0, 0), (0, LD - 2 * L2))).reshape(B, 8, 128)
    wvec = jnp.concatenate([
        jnp.full((SL,), 1, jnp.int32),
        jnp.full((DL,), 65536, jnp.int32),
        jnp.zeros((LP - L2,), jnp.int32),
    ]).reshape(4, 128)
    zvec = jnp.zeros((4, 128), jnp.int32)
    a = _sc_count(ids_pad, ids_dup, wvec, zvec).reshape(B, 256, 4)

    w1 = W1[0, :]
    z = jnp.zeros((2 * D,), jnp.float32)
    w1e = jnp.concatenate([w1, jnp.zeros((D,), jnp.float32)])
    w1o = jnp.concatenate([jnp.zeros((D,), jnp.float32), w1])
    m1a = jnp.stack([w1e, w1o, z, z])
    m1b = jnp.stack([z, z, w1e, w1o])
    b1p = jnp.concatenate([b1, b1]).reshape(1, 128)
    w2dd = jnp.block([[W2, jnp.zeros((D, D), jnp.float32)],
                      [jnp.zeros((D, D), jnp.float32), W2]])
    bb2 = (2.0 * jnp.concatenate([b2, b2])).reshape(1, 128)

    srcp, dstp = _tc_encode(a, m1a, m1b, b1p, w2dd, bb2)
    return srcp.reshape(B, SL, D), dstp.reshape(B, DL, D)
